# baseline (speedup 1.0000x reference)
import functools

import jax
import jax.numpy as jnp
from jax import lax
from jax.experimental import pallas as pl
from jax.experimental.pallas import tpu as pltpu
from jax.experimental.pallas import tpu_sc as plsc

_NUM_CLASSES = 1000000
_HIDDEN = 64
_BATCH = 16384

_NW = 32
_B_PER_W = _BATCH // _NW
_GRP = 16
_NITER = _B_PER_W // _GRP


def _embed_body(labels_hbm, drop_hbm, table_hbm, out_hbm,
                lab_v, drop_v, idx_v, cfg_v, rows_v, sem):
    wid = lax.axis_index("s") * 2 + lax.axis_index("c")
    base = wid * _B_PER_W

    cp_lab = pltpu.make_async_copy(
        labels_hbm.at[pl.ds(base, _B_PER_W)], lab_v, sem)
    cp_drop = pltpu.make_async_copy(
        drop_hbm.at[pl.ds(base, _B_PER_W)], drop_v, sem)
    cp_lab.start()
    cp_drop.start()
    cp_cfg = pltpu.make_async_copy(
        table_hbm.at[pl.ds(_NUM_CLASSES, 1), :], cfg_v, sem)
    cp_cfg.start()
    cp_cfg.wait()
    cp_lab.wait()
    cp_drop.wait()

    for i in range(_B_PER_W // 16):
        sl = pl.ds(i * 16, 16)
        idx_v[sl] = jnp.where(drop_v[sl] == 1,
                              jnp.full((16,), _NUM_CLASSES, jnp.int32),
                              lab_v[sl])

    cfg_row = cfg_v.at[0]
    cfg_regs = [cfg_row[pl.ds(k * 16, 16)] for k in range(4)]

    @pl.loop(0, _NITER)
    def _fill(g):
        row0 = g * _GRP
        dvec = drop_v[pl.ds(row0, 16)]
        for j in range(_GRP):
            p = row0 + j

            @pl.when(dvec[j] == 1)
            def _():
                for k in range(4):
                    rows_v.at[p][pl.ds(k * 16, 16)] = cfg_regs[k]

    @pl.loop(0, _NITER)
    def _gather(g):
        row0 = g * _GRP
        vec = idx_v[pl.ds(row0, 16)]
        dvec = drop_v[pl.ds(row0, 16)]
        for phase in range(2):
            for j in range(_GRP):
                ok = dvec[j] == 0
                cp = pltpu.make_async_copy(
                    table_hbm.at[pl.ds(vec[j], 1), :],
                    rows_v.at[pl.ds(row0 + j, 1), :],
                    sem,
                )

                @pl.when(ok)
                def _():
                    if phase == 0:
                        cp.start()
                    else:
                        cp.wait()

    pltpu.sync_copy(rows_v, out_hbm.at[pl.ds(base, _B_PER_W)])


_BLK = 512
_GRID = (_NUM_CLASSES + 1 + _BLK - 1) // _BLK


def _transpose_body(tt_ref, out_ref):
    out_ref[...] = tt_ref[...].T


_tc_transpose = pl.pallas_call(
    _transpose_body,
    grid=(_GRID,),
    in_specs=[pl.BlockSpec((_HIDDEN, _BLK), lambda j: (0, j))],
    out_specs=pl.BlockSpec((_BLK, _HIDDEN), lambda j: (j, 0)),
    out_shape=jax.ShapeDtypeStruct((_NUM_CLASSES + 1, _HIDDEN), jnp.float32),
)


@jax.jit
def _embed(labels, force_drop_ids, table):
    table_rm = _tc_transpose(jnp.swapaxes(table, 0, 1))
    mesh = plsc.VectorSubcoreMesh(core_axis_name="c", subcore_axis_name="s")
    f = functools.partial(
        pl.kernel,
        mesh=mesh,
        out_type=jax.ShapeDtypeStruct((_BATCH, _HIDDEN), jnp.float32),
        scratch_types=[
            pltpu.VMEM((_B_PER_W,), jnp.int32),
            pltpu.VMEM((_B_PER_W,), jnp.int32),
            pltpu.VMEM((_B_PER_W,), jnp.int32),
            pltpu.VMEM((1, _HIDDEN), jnp.float32),
            pltpu.VMEM((_B_PER_W, _HIDDEN), jnp.float32),
            pltpu.SemaphoreType.DMA,
        ],
    )(_embed_body)
    return f(labels, force_drop_ids, table_rm)


def kernel(labels, train, force_drop_ids, table):
    del train
    return _embed(labels.reshape(-1), force_drop_ids, table)


# speedup vs baseline: 3.2520x; 3.2520x over previous
import functools

import jax
import jax.numpy as jnp
from jax import lax
from jax.experimental import pallas as pl
from jax.experimental.pallas import tpu as pltpu
from jax.experimental.pallas import tpu_sc as plsc

_NUM_CLASSES = 1000000
_HIDDEN = 64
_BATCH = 16384

_NW = 32
_B_PER_W = _BATCH // _NW
_GRP = 16
_NITER = _B_PER_W // _GRP


def _embed_body(labels_hbm, drop_hbm, table_hbm, out_hbm,
                lab_v, drop_v, idx_v, cfg_v, rows_v, sem):
    wid = lax.axis_index("s") * 2 + lax.axis_index("c")
    base = wid * _B_PER_W

    cp_lab = pltpu.make_async_copy(
        labels_hbm.at[pl.ds(base, _B_PER_W)], lab_v, sem)
    cp_drop = pltpu.make_async_copy(
        drop_hbm.at[pl.ds(base, _B_PER_W)], drop_v, sem)
    cp_lab.start()
    cp_drop.start()
    cp_cfg = pltpu.make_async_copy(
        table_hbm.at[pl.ds(_NUM_CLASSES, 1), :], cfg_v, sem)
    cp_cfg.start()
    cp_cfg.wait()
    cp_lab.wait()
    cp_drop.wait()

    for i in range(_B_PER_W // 16):
        sl = pl.ds(i * 16, 16)
        idx_v[sl] = jnp.where(drop_v[sl] == 1,
                              jnp.full((16,), _NUM_CLASSES, jnp.int32),
                              lab_v[sl])

    cfg_row = cfg_v.at[0]
    cfg_regs = [cfg_row[pl.ds(k * 16, 16)] for k in range(4)]

    @pl.loop(0, _NITER)
    def _fill(g):
        row0 = g * _GRP
        dvec = drop_v[pl.ds(row0, 16)]
        for j in range(_GRP):
            p = row0 + j

            @pl.when(dvec[j] == 1)
            def _():
                for k in range(4):
                    rows_v.at[p][pl.ds(k * 16, 16)] = cfg_regs[k]

    @pl.loop(0, _NITER)
    def _gather(g):
        row0 = g * _GRP
        vec = idx_v[pl.ds(row0, 16)]
        dvec = drop_v[pl.ds(row0, 16)]
        for phase in range(2):
            for j in range(_GRP):
                ok = dvec[j] == 0
                cp = pltpu.make_async_copy(
                    table_hbm.at[pl.ds(vec[j], 1), :],
                    rows_v.at[pl.ds(row0 + j, 1), :],
                    sem,
                )

                @pl.when(ok)
                def _():
                    if phase == 0:
                        cp.start()
                    else:
                        cp.wait()

    pltpu.sync_copy(rows_v, out_hbm.at[pl.ds(base, _B_PER_W)])


@jax.jit
def _embed(labels, force_drop_ids, table):
    mesh = plsc.VectorSubcoreMesh(core_axis_name="c", subcore_axis_name="s")
    f = functools.partial(
        pl.kernel,
        mesh=mesh,
        out_type=jax.ShapeDtypeStruct((_BATCH, _HIDDEN), jnp.float32),
        scratch_types=[
            pltpu.VMEM((_B_PER_W,), jnp.int32),
            pltpu.VMEM((_B_PER_W,), jnp.int32),
            pltpu.VMEM((_B_PER_W,), jnp.int32),
            pltpu.VMEM((1, _HIDDEN), jnp.float32),
            pltpu.VMEM((_B_PER_W, _HIDDEN), jnp.float32),
            pltpu.SemaphoreType.DMA,
        ],
    )(_embed_body)
    return f(labels, force_drop_ids, table)


def kernel(labels, train, force_drop_ids, table):
    del train
    return _embed(labels.reshape(-1), force_drop_ids, table)
